# baseline (speedup 1.0000x reference)
import numpy as np
import jax
import jax.numpy as jnp
from jax import lax
from jax.experimental import pallas as pl
from jax.experimental.pallas import tpu as pltpu, tpu_sc as plsc

_RATIO = 0.7
_DIM_OUT = 128
_N_CENTER = 64
_KS = 16
_RADIUS = 0.4
_SIGMA = 0.1
_KA = 12
_M = 2048
_B = 2
_BC = _B * _N_CENTER
_NJ = _KA * _KS
_NCORES = 2
_NSUB = 16
_NW = _NCORES * _NSUB
_CPW = _BC // _NW
_NCHUNK = _M // 16


def _fib_sphere(n, r):
    i = np.arange(n, dtype=np.float64)
    phi = np.pi * (3.0 - np.sqrt(5.0))
    y = 1.0 - 2.0 * (i + 0.5) / n
    rad = np.sqrt(np.maximum(0.0, 1.0 - y * y))
    th = phi * i
    return (np.stack([np.cos(th) * rad, y, np.sin(th) * rad], axis=-1) * r).astype(np.float32)


def _mk_anchors(n):
    rng = np.random.RandomState(0)
    out = []
    for _ in range(n):
        a = rng.randn(3, 3)
        q, rmat = np.linalg.qr(a)
        q = q * np.sign(np.diag(rmat))[None, :]
        if np.linalg.det(q) < 0:
            q[:, 0] = -q[:, 0]
        out.append(q)
    return np.stack(out).astype(np.float32)


_KPTS = _fib_sphere(_KS, _RATIO * _RADIUS)
_ANCHORS_NP = _mk_anchors(_KA)
_KERNELS_NP = np.transpose(_ANCHORS_NP @ _KPTS.T, (2, 0, 1))
_KCOL_NP = np.transpose(_KERNELS_NP, (1, 0, 2)).reshape(_NJ, 3)
_K2_NP = np.sum(_KCOL_NP * _KCOL_NP, axis=-1)


def _sc_body(fx_hbm, fy_hbm, fz_hbm, cx_hbm, cy_hbm, cz_hbm,
             kx_hbm, ky_hbm, kz_hbm, s_hbm,
             fxv, fyv, fzv, cxv, cyv, czv, kxv, kyv, kzv,
             xcb, ycb, zcb, wb, accv):
    inv_s = 1.0 / _SIGMA
    inv_2s = 1.0 / (2.0 * _SIGMA)
    r2 = _RADIUS * _RADIUS
    wid = lax.axis_index("s") * _NCORES + lax.axis_index("c")

    pltpu.sync_copy(fx_hbm, fxv)
    pltpu.sync_copy(fy_hbm, fyv)
    pltpu.sync_copy(fz_hbm, fzv)
    pltpu.sync_copy(cx_hbm, cxv.at[pl.ds(0, _BC)])
    pltpu.sync_copy(cy_hbm, cyv.at[pl.ds(0, _BC)])
    pltpu.sync_copy(cz_hbm, czv.at[pl.ds(0, _BC)])
    pltpu.sync_copy(kx_hbm, kxv)
    pltpu.sync_copy(ky_hbm, kyv)
    pltpu.sync_copy(kz_hbm, kzv)

    lanes = lax.iota(jnp.int32, 16)

    def center_body(ci, _):
        c = wid * _CPW + ci
        cx = cxv[pl.ds(c, 16)][0]
        cy = cyv[pl.ds(c, 16)][0]
        cz = czv[pl.ds(c, 16)][0]

        def chunk_body(v, cursor):
            base = v * 16
            px = fxv[pl.ds(base, 16)]
            py = fyv[pl.ds(base, 16)]
            pz = fzv[pl.ds(base, 16)]
            dx = px - cx
            dy = py - cy
            dz = pz - cz
            d2 = dx * dx + dy * dy + dz * dz
            mask = d2 < r2
            w = jnp.exp(d2 * (-inv_2s))
            pos = cursor + plsc.cumsum(
                jnp.where(mask, jnp.int32(1), jnp.int32(0))) - 1
            plsc.store_scatter(xcb, [pos], px, mask=mask)
            plsc.store_scatter(ycb, [pos], py, mask=mask)
            plsc.store_scatter(zcb, [pos], pz, mask=mask)
            plsc.store_scatter(wb, [pos], w, mask=mask)
            return cursor + plsc.all_reduce_population_count(mask)

        cursor = plsc.parallel_loop(
            0, _NCHUNK, 1, unroll=2,
            carry=jnp.zeros((16,), jnp.int32))(chunk_body)
        nn = lax.reduce_max(cursor, (0,))

        kregs = tuple(kv[pl.ds(jv * 16, 16)]
                      for jv in range(_NJ // 16)
                      for kv in (kxv, kyv, kzv))

        def entry_body(e, acc):
            x = xcb[pl.ds(e, 16)][0]
            y = ycb[pl.ds(e, 16)][0]
            z = zcb[pl.ds(e, 16)][0]
            w = wb[pl.ds(e, 16)][0]
            out = []
            for jv in range(_NJ // 16):
                fk = (kregs[3 * jv] * x + kregs[3 * jv + 1] * y
                      + kregs[3 * jv + 2] * z)
                out.append(acc[jv] + w * jnp.exp(fk))
            return tuple(out)

        acc0 = tuple(jnp.zeros((16,), jnp.float32) for _ in range(_NJ // 16))
        acc = plsc.parallel_loop(0, nn, 1, unroll=4, carry=acc0)(entry_body)
        for jv in range(_NJ // 16):
            accv[pl.ds(jv * 16, 16)] = acc[jv]
        pltpu.sync_copy(accv, s_hbm.at[pl.ds(c * _NJ, _NJ)])
        return 0

    lax.fori_loop(0, _CPW, center_body, 0)


def _sc_accumulate(fx, fy, fz, cx, cy, cz, kx, ky, kz):
    mesh = plsc.VectorSubcoreMesh(core_axis_name="c", subcore_axis_name="s",
                                  num_cores=_NCORES, num_subcores=_NSUB)
    f = pl.kernel(
        _sc_body,
        out_type=jax.ShapeDtypeStruct((_BC * _NJ,), jnp.float32),
        mesh=mesh,
        compiler_params=pltpu.CompilerParams(needs_layout_passes=False),
        scratch_types=[
            pltpu.VMEM((_M,), jnp.float32),
            pltpu.VMEM((_M,), jnp.float32),
            pltpu.VMEM((_M,), jnp.float32),
            pltpu.VMEM((_BC + 16,), jnp.float32),
            pltpu.VMEM((_BC + 16,), jnp.float32),
            pltpu.VMEM((_BC + 16,), jnp.float32),
            pltpu.VMEM((_NJ,), jnp.float32),
            pltpu.VMEM((_NJ,), jnp.float32),
            pltpu.VMEM((_NJ,), jnp.float32),
            pltpu.VMEM((_M + 16,), jnp.float32),
            pltpu.VMEM((_M + 16,), jnp.float32),
            pltpu.VMEM((_M + 16,), jnp.float32),
            pltpu.VMEM((_M + 16,), jnp.float32),
            pltpu.VMEM((_NJ,), jnp.float32),
        ],
    )
    return f(fx, fy, fz, cx, cy, cz, kx, ky, kz)


def _tc_body(frag_ref, clouds_ref, w_ref, s_ref, kcolT_ref, k2_ref, out_ref):
    inv_s = 1.0 / _SIGMA
    inv_2s = 1.0 / (2.0 * _SIGMA)
    frag = frag_ref[:]
    kcolT = kcolT_ref[:]
    C = jnp.concatenate([jnp.transpose(clouds_ref[0]),
                         jnp.transpose(clouds_ref[1])], axis=0)
    CT = jnp.transpose(C)

    d0 = frag[:, 0:1] - CT[0:1, :]
    d1 = frag[:, 1:2] - CT[1:2, :]
    d2_ = frag[:, 2:3] - CT[2:3, :]
    d2c = d0 * d0 + d1 * d1 + d2_ * d2_
    mask = d2c < (_RADIUS * _RADIUS)
    nnT = jnp.sum(jnp.where(mask, 1.0, 0.0), axis=0, keepdims=True)
    nn = jnp.transpose(nnT)

    CK = (C[:, 0:1] * kcolT[0:1, :]
          + C[:, 1:2] * kcolT[1:2, :]
          + C[:, 2:3] * kcolT[2:3, :])
    g = jnp.exp(k2_ref[:] * (-inv_2s) - CK * inv_s)
    Ss = s_ref[:] * g / (nn + 1.0)

    wT = jnp.transpose(w_ref[:])
    for a in range(_KA):
        out_ref[:, a * _DIM_OUT:(a + 1) * _DIM_OUT] = jax.lax.dot_general(
            Ss[:, a * _KS:(a + 1) * _KS], wT, (((1,), (0,)), ((), ())),
            preferred_element_type=jnp.float32,
            precision=jax.lax.Precision.HIGHEST)


def kernel(frag, clouds, W):
    kcolT = jnp.asarray(_KCOL_NP.T)
    k2 = jnp.asarray(_K2_NP)[None, :]
    kx = jnp.asarray(_KCOL_NP[:, 0] / _SIGMA)
    ky = jnp.asarray(_KCOL_NP[:, 1] / _SIGMA)
    kz = jnp.asarray(_KCOL_NP[:, 2] / _SIGMA)
    C = jnp.transpose(clouds, (0, 2, 1)).reshape(_BC, 3)

    S = _sc_accumulate(frag[:, 0], frag[:, 1], frag[:, 2],
                       C[:, 0], C[:, 1], C[:, 2],
                       kx, ky, kz).reshape(_BC, _NJ)

    F = pl.pallas_call(
        _tc_body,
        out_shape=jax.ShapeDtypeStruct((_BC, _KA * _DIM_OUT), jnp.float32),
    )(frag, clouds, W, S, kcolT, k2)

    feats = F.reshape(_B, _N_CENTER, _KA, _DIM_OUT).transpose(0, 3, 1, 2)
    return clouds, feats, jnp.asarray(_ANCHORS_NP)


# speedup vs baseline: 1.8966x; 1.8966x over previous
import numpy as np
import jax
import jax.numpy as jnp
from jax import lax
from jax.experimental import pallas as pl
from jax.experimental.pallas import tpu as pltpu, tpu_sc as plsc

_RATIO = 0.7
_DIM_OUT = 128
_N_CENTER = 64
_KS = 16
_RADIUS = 0.4
_SIGMA = 0.1
_KA = 12
_M = 2048
_B = 2
_BC = _B * _N_CENTER
_NJ = _KA * _KS
_NCORES = 2
_NSUB = 16
_NW = _NCORES * _NSUB
_CPW = _BC // _NW
_NCHUNK = _M // 16


def _fib_sphere(n, r):
    i = np.arange(n, dtype=np.float64)
    phi = np.pi * (3.0 - np.sqrt(5.0))
    y = 1.0 - 2.0 * (i + 0.5) / n
    rad = np.sqrt(np.maximum(0.0, 1.0 - y * y))
    th = phi * i
    return (np.stack([np.cos(th) * rad, y, np.sin(th) * rad], axis=-1) * r).astype(np.float32)


def _mk_anchors(n):
    rng = np.random.RandomState(0)
    out = []
    for _ in range(n):
        a = rng.randn(3, 3)
        q, rmat = np.linalg.qr(a)
        q = q * np.sign(np.diag(rmat))[None, :]
        if np.linalg.det(q) < 0:
            q[:, 0] = -q[:, 0]
        out.append(q)
    return np.stack(out).astype(np.float32)


_KPTS = _fib_sphere(_KS, _RATIO * _RADIUS)
_ANCHORS_NP = _mk_anchors(_KA)
_KERNELS_NP = np.transpose(_ANCHORS_NP @ _KPTS.T, (2, 0, 1))
_KCOL_NP = np.transpose(_KERNELS_NP, (1, 0, 2)).reshape(_NJ, 3)
_K2_NP = np.sum(_KCOL_NP * _KCOL_NP, axis=-1)


def _sc_body(fx_hbm, fy_hbm, fz_hbm, cx_hbm, cy_hbm, cz_hbm,
             kx_hbm, ky_hbm, kz_hbm, s_hbm,
             fxv, fyv, fzv, cxv, cyv, czv, kxv, kyv, kzv,
             xcb, ycb, zcb, wb, accv):
    inv_s = 1.0 / _SIGMA
    inv_2s = 1.0 / (2.0 * _SIGMA)
    r2 = _RADIUS * _RADIUS
    wid = lax.axis_index("s") * _NCORES + lax.axis_index("c")

    pltpu.sync_copy(fx_hbm, fxv)
    pltpu.sync_copy(fy_hbm, fyv)
    pltpu.sync_copy(fz_hbm, fzv)
    pltpu.sync_copy(cx_hbm, cxv.at[pl.ds(0, _BC)])
    pltpu.sync_copy(cy_hbm, cyv.at[pl.ds(0, _BC)])
    pltpu.sync_copy(cz_hbm, czv.at[pl.ds(0, _BC)])
    pltpu.sync_copy(kx_hbm, kxv)
    pltpu.sync_copy(ky_hbm, kyv)
    pltpu.sync_copy(kz_hbm, kzv)

    lanes = lax.iota(jnp.int32, 16)

    def center_body(ci, _):
        c = wid * _CPW + ci
        cx = cxv[pl.ds(c, 16)][0]
        cy = cyv[pl.ds(c, 16)][0]
        cz = czv[pl.ds(c, 16)][0]

        def chunk_body(v, cursor):
            base = v * 16
            px = fxv[pl.ds(base, 16)]
            py = fyv[pl.ds(base, 16)]
            pz = fzv[pl.ds(base, 16)]
            dx = px - cx
            dy = py - cy
            dz = pz - cz
            d2 = dx * dx + dy * dy + dz * dz
            mask = d2 < r2
            w = jnp.exp(d2 * (-inv_2s))
            pos = cursor + plsc.cumsum(
                jnp.where(mask, jnp.int32(1), jnp.int32(0))) - 1
            plsc.store_scatter(xcb, [pos], px, mask=mask)
            plsc.store_scatter(ycb, [pos], py, mask=mask)
            plsc.store_scatter(zcb, [pos], pz, mask=mask)
            plsc.store_scatter(wb, [pos], w, mask=mask)
            return cursor + plsc.all_reduce_population_count(mask)

        cursor = lax.fori_loop(0, _NCHUNK, chunk_body,
                               jnp.zeros((16,), jnp.int32))
        nn = lax.reduce_max(cursor, (0,))

        kregs = tuple(kv[pl.ds(jv * 16, 16)]
                      for jv in range(_NJ // 16)
                      for kv in (kxv, kyv, kzv))

        def entry_body(e, acc):
            ev = jnp.full((16,), e, jnp.int32)
            x = plsc.load_gather(xcb, [ev])
            y = plsc.load_gather(ycb, [ev])
            z = plsc.load_gather(zcb, [ev])
            w = plsc.load_gather(wb, [ev])
            out = []
            for jv in range(_NJ // 16):
                fk = (kregs[3 * jv] * x + kregs[3 * jv + 1] * y
                      + kregs[3 * jv + 2] * z)
                out.append(acc[jv] + w * jnp.exp(fk))
            return tuple(out)

        acc0 = tuple(jnp.zeros((16,), jnp.float32) for _ in range(_NJ // 16))
        acc = lax.fori_loop(0, nn, entry_body, acc0)
        for jv in range(_NJ // 16):
            accv[pl.ds(jv * 16, 16)] = acc[jv]
        pltpu.sync_copy(accv, s_hbm.at[pl.ds(c * _NJ, _NJ)])
        return 0

    lax.fori_loop(0, _CPW, center_body, 0)


def _sc_accumulate(fx, fy, fz, cx, cy, cz, kx, ky, kz):
    mesh = plsc.VectorSubcoreMesh(core_axis_name="c", subcore_axis_name="s",
                                  num_cores=_NCORES, num_subcores=_NSUB)
    f = pl.kernel(
        _sc_body,
        out_type=jax.ShapeDtypeStruct((_BC * _NJ,), jnp.float32),
        mesh=mesh,
        compiler_params=pltpu.CompilerParams(needs_layout_passes=False),
        scratch_types=[
            pltpu.VMEM((_M,), jnp.float32),
            pltpu.VMEM((_M,), jnp.float32),
            pltpu.VMEM((_M,), jnp.float32),
            pltpu.VMEM((_BC + 16,), jnp.float32),
            pltpu.VMEM((_BC + 16,), jnp.float32),
            pltpu.VMEM((_BC + 16,), jnp.float32),
            pltpu.VMEM((_NJ,), jnp.float32),
            pltpu.VMEM((_NJ,), jnp.float32),
            pltpu.VMEM((_NJ,), jnp.float32),
            pltpu.VMEM((_M + 16,), jnp.float32),
            pltpu.VMEM((_M + 16,), jnp.float32),
            pltpu.VMEM((_M + 16,), jnp.float32),
            pltpu.VMEM((_M + 16,), jnp.float32),
            pltpu.VMEM((_NJ,), jnp.float32),
        ],
    )
    return f(fx, fy, fz, cx, cy, cz, kx, ky, kz)


def _tc_body(frag_ref, clouds_ref, w_ref, s_ref, kcolT_ref, k2_ref, out_ref):
    inv_s = 1.0 / _SIGMA
    inv_2s = 1.0 / (2.0 * _SIGMA)
    frag = frag_ref[:]
    kcolT = kcolT_ref[:]
    C = jnp.concatenate([jnp.transpose(clouds_ref[0]),
                         jnp.transpose(clouds_ref[1])], axis=0)
    CT = jnp.transpose(C)

    d0 = frag[:, 0:1] - CT[0:1, :]
    d1 = frag[:, 1:2] - CT[1:2, :]
    d2_ = frag[:, 2:3] - CT[2:3, :]
    d2c = d0 * d0 + d1 * d1 + d2_ * d2_
    mask = d2c < (_RADIUS * _RADIUS)
    nnT = jnp.sum(jnp.where(mask, 1.0, 0.0), axis=0, keepdims=True)
    nn = jnp.transpose(nnT)

    CK = (C[:, 0:1] * kcolT[0:1, :]
          + C[:, 1:2] * kcolT[1:2, :]
          + C[:, 2:3] * kcolT[2:3, :])
    g = jnp.exp(k2_ref[:] * (-inv_2s) - CK * inv_s)
    Ss = s_ref[:] * g / (nn + 1.0)

    wT = jnp.transpose(w_ref[:])
    for a in range(_KA):
        out_ref[:, a * _DIM_OUT:(a + 1) * _DIM_OUT] = jax.lax.dot_general(
            Ss[:, a * _KS:(a + 1) * _KS], wT, (((1,), (0,)), ((), ())),
            preferred_element_type=jnp.float32,
            precision=jax.lax.Precision.HIGHEST)


def kernel(frag, clouds, W):
    kcolT = jnp.asarray(_KCOL_NP.T)
    k2 = jnp.asarray(_K2_NP)[None, :]
    kx = jnp.asarray(_KCOL_NP[:, 0] / _SIGMA)
    ky = jnp.asarray(_KCOL_NP[:, 1] / _SIGMA)
    kz = jnp.asarray(_KCOL_NP[:, 2] / _SIGMA)
    C = jnp.transpose(clouds, (0, 2, 1)).reshape(_BC, 3)

    S = _sc_accumulate(frag[:, 0], frag[:, 1], frag[:, 2],
                       C[:, 0], C[:, 1], C[:, 2],
                       kx, ky, kz).reshape(_BC, _NJ)

    F = pl.pallas_call(
        _tc_body,
        out_shape=jax.ShapeDtypeStruct((_BC, _KA * _DIM_OUT), jnp.float32),
    )(frag, clouds, W, S, kcolT, k2)

    feats = F.reshape(_B, _N_CENTER, _KA, _DIM_OUT).transpose(0, 3, 1, 2)
    return clouds, feats, jnp.asarray(_ANCHORS_NP)
